# baseline (speedup 1.0000x reference)
import functools

import jax
import jax.numpy as jnp
from jax import lax
from jax.experimental import pallas as pl
from jax.experimental.pallas import tpu as pltpu
from jax.experimental.pallas import tpu_sc as plsc

VOCAB = 100000
EMB = 128
CTX = 200

_ROWS_PER_WORKER = 8
_NUM_ACTIVE = CTX // _ROWS_PER_WORKER

NW = 32

BV = 16384
VSC = 32768
BSTART = VSC // BV
NBTC = -(-(VOCAB - VSC) // BV)
TCPAD = NBTC * BV

RPT = VSC // NW
CHUNK = 256
NCHUNK = RPT // CHUNK
GPC = CHUNK // 16

_NEG = -1e30


def _sc_gather_kernel(idx_hbm, table_hbm, out_hbm, idx_v, rows_v, acc_v, sem):
    nc = plsc.get_sparse_core_info().num_cores
    wid = lax.axis_index("s") * nc + lax.axis_index("c")

    @pl.when(wid < _NUM_ACTIVE)
    def _gather():
        pltpu.sync_copy(idx_hbm.at[pl.ds(wid * _ROWS_PER_WORKER, _ROWS_PER_WORKER)], idx_v)
        pltpu.async_copy(table_hbm.at[idx_v], rows_v, sem).wait()
        for c in range(EMB // 16):
            acc = rows_v[0, pl.ds(c * 16, 16)]
            for r in range(1, _ROWS_PER_WORKER):
                acc = acc + rows_v[r, pl.ds(c * 16, 16)]
            acc_v[pl.ds(c * 16, 16)] = acc

    @pl.when(wid >= _NUM_ACTIVE)
    def _zero():
        for c in range(EMB // 16):
            acc_v[pl.ds(c * 16, 16)] = jnp.zeros((16,), jnp.float32)

    pltpu.sync_copy(acc_v, out_hbm.at[wid])


def _sc_gather(context_idxs, emb_table):
    mesh = plsc.VectorSubcoreMesh(core_axis_name="c", subcore_axis_name="s")
    kern = functools.partial(
        pl.kernel,
        mesh=mesh,
        out_type=jax.ShapeDtypeStruct((NW, EMB), jnp.float32),
        scratch_types=[
            pltpu.VMEM((_ROWS_PER_WORKER,), jnp.int32),
            pltpu.VMEM((_ROWS_PER_WORKER, EMB), jnp.float32),
            pltpu.VMEM((EMB,), jnp.float32),
            pltpu.SemaphoreType.DMA,
        ],
    )(_sc_gather_kernel)
    return kern(context_idxs, emb_table)


_GDN = lax.GatherDimensionNumbers(
    offset_dims=(), collapsed_slice_dims=(0,), start_index_map=(0,))


def _permute(v, perm):
    return lax.gather(
        v, perm[:, None], _GDN, (1,),
        indices_are_sorted=False, unique_indices=False,
        mode=lax.GatherScatterMode.PROMISE_IN_BOUNDS)


def _sc_matvec_kernel(part_hbm, w_hbm, b_hbm, out_hbm,
                      pbuf, bbuf, lbuf, wb0, wb1, sem0, sem1):
    nc = plsc.get_sparse_core_info().num_cores
    wid = lax.axis_index("s") * nc + lax.axis_index("c")
    row0 = wid * RPT

    pltpu.sync_copy(part_hbm, pbuf)
    vvec = []
    for c in range(EMB // 16):
        acc = pbuf[0, pl.ds(c * 16, 16)]
        for r in range(1, NW):
            acc = acc + pbuf[r, pl.ds(c * 16, 16)]
        vvec.append(acc * (1.0 / CTX))

    pltpu.sync_copy(b_hbm.at[pl.ds(row0, RPT)], bbuf)

    wbufs = [wb0, wb1]
    sems = [sem0, sem1]
    copies = [None, None]
    copies[0] = pltpu.async_copy(w_hbm.at[pl.ds(row0 * EMB, CHUNK * EMB)], wb0, sem0)

    lane = lax.broadcasted_iota(jnp.int32, (16,), 0)
    perms = [lane ^ sh for sh in (8, 4, 2, 1)]
    for k in range(NCHUNK):
        if k + 1 < NCHUNK:
            copies[(k + 1) % 2] = pltpu.async_copy(
                w_hbm.at[pl.ds((row0 + (k + 1) * CHUNK) * EMB, CHUNK * EMB)],
                wbufs[(k + 1) % 2], sems[(k + 1) % 2])
        copies[k % 2].wait()
        wb = wbufs[k % 2]
        coff = k * CHUNK

        def group_body(g, carry, wb=wb, coff=coff):
            out16 = bbuf[pl.ds(coff + g * 16, 16)]
            base = g * (16 * EMB)
            for r in range(16):
                off = base + r * EMB
                a0 = wb[pl.ds(off, 16)] * vvec[0]
                a1 = wb[pl.ds(off + 16, 16)] * vvec[1]
                a2 = wb[pl.ds(off + 32, 16)] * vvec[2]
                a3 = wb[pl.ds(off + 48, 16)] * vvec[3]
                a0 = a0 + wb[pl.ds(off + 64, 16)] * vvec[4]
                a1 = a1 + wb[pl.ds(off + 80, 16)] * vvec[5]
                a2 = a2 + wb[pl.ds(off + 96, 16)] * vvec[6]
                a3 = a3 + wb[pl.ds(off + 112, 16)] * vvec[7]
                s = (a0 + a1) + (a2 + a3)
                for p in perms:
                    s = s + _permute(s, p)
                out16 = jnp.where(lane == r, s, out16)
            lbuf[pl.ds(coff + g * 16, 16)] = out16
            return carry

        lax.fori_loop(0, GPC, group_body, 0)

    pltpu.sync_copy(lbuf, out_hbm.at[pl.ds(row0, RPT)])


def _sc_matvec(w_flat, partials, b):
    mesh = plsc.VectorSubcoreMesh(core_axis_name="c", subcore_axis_name="s")
    kern = functools.partial(
        pl.kernel,
        mesh=mesh,
        out_type=jax.ShapeDtypeStruct((VSC,), jnp.float32),
        scratch_types=[
            pltpu.VMEM((NW, EMB), jnp.float32),
            pltpu.VMEM((RPT,), jnp.float32),
            pltpu.VMEM((RPT,), jnp.float32),
            pltpu.VMEM((CHUNK * EMB,), jnp.float32),
            pltpu.VMEM((CHUNK * EMB,), jnp.float32),
            pltpu.SemaphoreType.DMA,
            pltpu.SemaphoreType.DMA,
        ],
    )(_sc_matvec_kernel)
    return kern(partials, w_flat, b)


def _tc_kernel(part_ref, w_ref, b_ref, out_ref):
    i = pl.program_id(0)
    v = jnp.sum(part_ref[...], axis=0, keepdims=True) * (1.0 / CTX)
    logits = lax.dot_general(
        v.astype(jnp.bfloat16), w_ref[...].astype(jnp.bfloat16),
        (((1,), (1,)), ((), ())),
        preferred_element_type=jnp.float32,
    ) + b_ref[...]
    col = VSC + i * BV + lax.broadcasted_iota(jnp.int32, (1, BV), 1)
    logits = jnp.where(col < VOCAB, logits, _NEG)
    out_ref[0, pl.ds(i * BV, BV)] = logits[0, :]


def _tc_logits(partials, W, b2d):
    return pl.pallas_call(
        _tc_kernel,
        grid=(NBTC,),
        in_specs=[
            pl.BlockSpec((NW, EMB), lambda i: (0, 0)),
            pl.BlockSpec((BV, EMB), lambda i: (BSTART + i, 0)),
            pl.BlockSpec((1, BV), lambda i: (0, BSTART + i)),
        ],
        out_specs=pl.BlockSpec((1, TCPAD), lambda i: (0, 0)),
        out_shape=jax.ShapeDtypeStruct((1, TCPAD), jnp.float32),
    )(partials, W, b2d)


def _merge_kernel(lsc_ref, ltc_ref, out_ref):
    lsc = lsc_ref[...]
    ltc = ltc_ref[...]
    m = jnp.maximum(jnp.max(lsc), jnp.max(ltc))
    s = jnp.sum(jnp.exp(lsc - m)) + jnp.sum(jnp.exp(ltc - m))
    lz_sc = m + jnp.log(jnp.full((1, VSC), s, jnp.float32))
    lz_tc = m + jnp.log(jnp.full((1, TCPAD), s, jnp.float32))
    out_ref[0, pl.ds(0, VSC)] = (lsc - lz_sc)[0, :]
    out_ref[0, pl.ds(VSC, TCPAD)] = (ltc - lz_tc)[0, :]


def _merge(lsc2d, ltc):
    return pl.pallas_call(
        _merge_kernel,
        grid=(1,),
        in_specs=[
            pl.BlockSpec((1, VSC), lambda i: (0, 0)),
            pl.BlockSpec((1, TCPAD), lambda i: (0, 0)),
        ],
        out_specs=pl.BlockSpec((1, VSC + TCPAD), lambda i: (0, 0)),
        out_shape=jax.ShapeDtypeStruct((1, VOCAB), jnp.float32),
    )(lsc2d, ltc)


def kernel(context_idxs, emb_table, W, b):
    idx = context_idxs.astype(jnp.int32)
    partials = _sc_gather(idx, emb_table)
    lsc = _sc_matvec(W.reshape(-1), partials, b)
    ltc = _tc_logits(partials, W, b.reshape(1, VOCAB))
    return _merge(lsc.reshape(1, VSC), ltc)


# speedup vs baseline: 1.0732x; 1.0732x over previous
import functools

import jax
import jax.numpy as jnp
from jax import lax
from jax.experimental import pallas as pl
from jax.experimental.pallas import tpu as pltpu
from jax.experimental.pallas import tpu_sc as plsc

VOCAB = 100000
EMB = 128
CTX = 200

_ROWS_PER_WORKER = 8
_NUM_ACTIVE = CTX // _ROWS_PER_WORKER

NW = 32

BV = 16384
VSC = 16384
BSTART = VSC // BV
NBTC = -(-(VOCAB - VSC) // BV)
TCPAD = NBTC * BV

RPT = VSC // NW
CHUNK = 256
NCHUNK = RPT // CHUNK
GPC = CHUNK // 16

_NEG = -1e30


def _sc_gather_kernel(idx_hbm, table_hbm, out_hbm, idx_v, rows_v, acc_v, sem):
    nc = plsc.get_sparse_core_info().num_cores
    wid = lax.axis_index("s") * nc + lax.axis_index("c")

    @pl.when(wid < _NUM_ACTIVE)
    def _gather():
        pltpu.sync_copy(idx_hbm.at[pl.ds(wid * _ROWS_PER_WORKER, _ROWS_PER_WORKER)], idx_v)
        pltpu.async_copy(table_hbm.at[idx_v], rows_v, sem).wait()
        for c in range(EMB // 16):
            acc = rows_v[0, pl.ds(c * 16, 16)]
            for r in range(1, _ROWS_PER_WORKER):
                acc = acc + rows_v[r, pl.ds(c * 16, 16)]
            acc_v[pl.ds(c * 16, 16)] = acc

    @pl.when(wid >= _NUM_ACTIVE)
    def _zero():
        for c in range(EMB // 16):
            acc_v[pl.ds(c * 16, 16)] = jnp.zeros((16,), jnp.float32)

    pltpu.sync_copy(acc_v, out_hbm.at[wid])


def _sc_gather(context_idxs, emb_table):
    mesh = plsc.VectorSubcoreMesh(core_axis_name="c", subcore_axis_name="s")
    kern = functools.partial(
        pl.kernel,
        mesh=mesh,
        out_type=jax.ShapeDtypeStruct((NW, EMB), jnp.float32),
        scratch_types=[
            pltpu.VMEM((_ROWS_PER_WORKER,), jnp.int32),
            pltpu.VMEM((_ROWS_PER_WORKER, EMB), jnp.float32),
            pltpu.VMEM((EMB,), jnp.float32),
            pltpu.SemaphoreType.DMA,
        ],
    )(_sc_gather_kernel)
    return kern(context_idxs, emb_table)


_GDN = lax.GatherDimensionNumbers(
    offset_dims=(), collapsed_slice_dims=(0,), start_index_map=(0,))


def _permute(v, perm):
    return lax.gather(
        v, perm[:, None], _GDN, (1,),
        indices_are_sorted=False, unique_indices=False,
        mode=lax.GatherScatterMode.PROMISE_IN_BOUNDS)


def _sc_matvec_kernel(part_hbm, w_hbm, b_hbm, out_hbm,
                      pbuf, bbuf, lbuf, wb0, wb1, sem0, sem1):
    nc = plsc.get_sparse_core_info().num_cores
    wid = lax.axis_index("s") * nc + lax.axis_index("c")
    row0 = wid * RPT

    pltpu.sync_copy(part_hbm, pbuf)
    vvec = []
    for c in range(EMB // 16):
        acc = pbuf[0, pl.ds(c * 16, 16)]
        for r in range(1, NW):
            acc = acc + pbuf[r, pl.ds(c * 16, 16)]
        vvec.append(acc * (1.0 / CTX))

    pltpu.sync_copy(b_hbm.at[pl.ds(row0, RPT)], bbuf)

    wbufs = [wb0, wb1]
    sems = [sem0, sem1]
    copies = [None, None]
    copies[0] = pltpu.async_copy(w_hbm.at[pl.ds(row0 * EMB, CHUNK * EMB)], wb0, sem0)

    lane = lax.broadcasted_iota(jnp.int32, (16,), 0)
    perms = [lane ^ sh for sh in (8, 4, 2, 1)]
    for k in range(NCHUNK):
        if k + 1 < NCHUNK:
            copies[(k + 1) % 2] = pltpu.async_copy(
                w_hbm.at[pl.ds((row0 + (k + 1) * CHUNK) * EMB, CHUNK * EMB)],
                wbufs[(k + 1) % 2], sems[(k + 1) % 2])
        copies[k % 2].wait()
        wb = wbufs[k % 2]
        coff = k * CHUNK

        def group_body(g, carry, wb=wb, coff=coff):
            out16 = bbuf[pl.ds(coff + g * 16, 16)]
            base = g * (16 * EMB)
            for r in range(16):
                off = base + r * EMB
                a0 = wb[pl.ds(off, 16)] * vvec[0]
                a1 = wb[pl.ds(off + 16, 16)] * vvec[1]
                a2 = wb[pl.ds(off + 32, 16)] * vvec[2]
                a3 = wb[pl.ds(off + 48, 16)] * vvec[3]
                a0 = a0 + wb[pl.ds(off + 64, 16)] * vvec[4]
                a1 = a1 + wb[pl.ds(off + 80, 16)] * vvec[5]
                a2 = a2 + wb[pl.ds(off + 96, 16)] * vvec[6]
                a3 = a3 + wb[pl.ds(off + 112, 16)] * vvec[7]
                s = (a0 + a1) + (a2 + a3)
                for p in perms:
                    s = s + _permute(s, p)
                out16 = jnp.where(lane == r, s, out16)
            lbuf[pl.ds(coff + g * 16, 16)] = out16
            return carry

        lax.fori_loop(0, GPC, group_body, 0)

    pltpu.sync_copy(lbuf, out_hbm.at[pl.ds(row0, RPT)])


def _sc_matvec(w_flat, partials, b):
    mesh = plsc.VectorSubcoreMesh(core_axis_name="c", subcore_axis_name="s")
    kern = functools.partial(
        pl.kernel,
        mesh=mesh,
        out_type=jax.ShapeDtypeStruct((VSC,), jnp.float32),
        scratch_types=[
            pltpu.VMEM((NW, EMB), jnp.float32),
            pltpu.VMEM((RPT,), jnp.float32),
            pltpu.VMEM((RPT,), jnp.float32),
            pltpu.VMEM((CHUNK * EMB,), jnp.float32),
            pltpu.VMEM((CHUNK * EMB,), jnp.float32),
            pltpu.SemaphoreType.DMA,
            pltpu.SemaphoreType.DMA,
        ],
    )(_sc_matvec_kernel)
    return kern(partials, w_flat, b)


def _tc_kernel(part_ref, w_ref, b_ref, out_ref):
    i = pl.program_id(0)
    v = jnp.sum(part_ref[...], axis=0, keepdims=True) * (1.0 / CTX)
    logits = lax.dot_general(
        v.astype(jnp.bfloat16), w_ref[...].astype(jnp.bfloat16),
        (((1,), (1,)), ((), ())),
        preferred_element_type=jnp.float32,
    ) + b_ref[...]
    col = VSC + i * BV + lax.broadcasted_iota(jnp.int32, (1, BV), 1)
    logits = jnp.where(col < VOCAB, logits, _NEG)
    out_ref[0, pl.ds(i * BV, BV)] = logits[0, :]


def _tc_logits(partials, W, b2d):
    return pl.pallas_call(
        _tc_kernel,
        grid=(NBTC,),
        in_specs=[
            pl.BlockSpec((NW, EMB), lambda i: (0, 0)),
            pl.BlockSpec((BV, EMB), lambda i: (BSTART + i, 0)),
            pl.BlockSpec((1, BV), lambda i: (0, BSTART + i)),
        ],
        out_specs=pl.BlockSpec((1, TCPAD), lambda i: (0, 0)),
        out_shape=jax.ShapeDtypeStruct((1, TCPAD), jnp.float32),
    )(partials, W, b2d)


def _merge_kernel(lsc_ref, ltc_ref, out_ref):
    lsc = lsc_ref[...]
    ltc = ltc_ref[...]
    m = jnp.maximum(jnp.max(lsc), jnp.max(ltc))
    s = jnp.sum(jnp.exp(lsc - m)) + jnp.sum(jnp.exp(ltc - m))
    lz_sc = m + jnp.log(jnp.full((1, VSC), s, jnp.float32))
    lz_tc = m + jnp.log(jnp.full((1, TCPAD), s, jnp.float32))
    out_ref[0, pl.ds(0, VSC)] = (lsc - lz_sc)[0, :]
    out_ref[0, pl.ds(VSC, TCPAD)] = (ltc - lz_tc)[0, :]


def _merge(lsc2d, ltc):
    return pl.pallas_call(
        _merge_kernel,
        grid=(1,),
        in_specs=[
            pl.BlockSpec((1, VSC), lambda i: (0, 0)),
            pl.BlockSpec((1, TCPAD), lambda i: (0, 0)),
        ],
        out_specs=pl.BlockSpec((1, VSC + TCPAD), lambda i: (0, 0)),
        out_shape=jax.ShapeDtypeStruct((1, VOCAB), jnp.float32),
    )(lsc2d, ltc)


def kernel(context_idxs, emb_table, W, b):
    idx = context_idxs.astype(jnp.int32)
    partials = _sc_gather(idx, emb_table)
    lsc = _sc_matvec(W.reshape(-1), partials, b)
    ltc = _tc_logits(partials, W, b.reshape(1, VOCAB))
    return _merge(lsc.reshape(1, VSC), ltc)


# speedup vs baseline: 1.2234x; 1.1399x over previous
import functools

import jax
import jax.numpy as jnp
from jax import lax
from jax.experimental import pallas as pl
from jax.experimental.pallas import tpu as pltpu
from jax.experimental.pallas import tpu_sc as plsc

VOCAB = 100000
EMB = 128
CTX = 200

_ROWS_PER_WORKER = 8
_NUM_ACTIVE = CTX // _ROWS_PER_WORKER

BV = 25088
NB = -(-VOCAB // BV)
VPAD = NB * BV

_NEG = -1e30


def _sc_gather_kernel(idx_hbm, table_hbm, out_hbm, idx_v, rows_v, acc_v, sem):
    nc = plsc.get_sparse_core_info().num_cores
    wid = lax.axis_index("s") * nc + lax.axis_index("c")

    @pl.when(wid < _NUM_ACTIVE)
    def _gather():
        pltpu.sync_copy(idx_hbm.at[pl.ds(wid * _ROWS_PER_WORKER, _ROWS_PER_WORKER)], idx_v)
        pltpu.async_copy(table_hbm.at[idx_v], rows_v, sem).wait()
        for c in range(EMB // 16):
            acc = rows_v[0, pl.ds(c * 16, 16)]
            for r in range(1, _ROWS_PER_WORKER):
                acc = acc + rows_v[r, pl.ds(c * 16, 16)]
            acc_v[pl.ds(c * 16, 16)] = acc

    @pl.when(wid >= _NUM_ACTIVE)
    def _zero():
        for c in range(EMB // 16):
            acc_v[pl.ds(c * 16, 16)] = jnp.zeros((16,), jnp.float32)

    pltpu.sync_copy(acc_v, out_hbm.at[wid])


def _sc_gather(context_idxs, emb_table):
    mesh = plsc.VectorSubcoreMesh(core_axis_name="c", subcore_axis_name="s")
    kern = functools.partial(
        pl.kernel,
        mesh=mesh,
        out_type=jax.ShapeDtypeStruct((32, EMB), jnp.float32),
        scratch_types=[
            pltpu.VMEM((_ROWS_PER_WORKER,), jnp.int32),
            pltpu.VMEM((_ROWS_PER_WORKER, EMB), jnp.float32),
            pltpu.VMEM((EMB,), jnp.float32),
            pltpu.SemaphoreType.DMA,
        ],
    )(_sc_gather_kernel)
    return kern(context_idxs, emb_table)


def _tc_kernel(part_ref, w_ref, b_ref, out_ref, m_ref):
    i = pl.program_id(0)

    @pl.when(i < NB)
    def _phase1():
        v = jnp.sum(part_ref[...], axis=0, keepdims=True) * (1.0 / CTX)
        logits = lax.dot_general(
            v.astype(jnp.bfloat16), w_ref[...].astype(jnp.bfloat16),
            (((1,), (1,)), ((), ())),
            preferred_element_type=jnp.float32,
        ) + b_ref[...]
        col = i * BV + lax.broadcasted_iota(jnp.int32, (1, BV), 1)
        logits = jnp.where(col < VOCAB, logits, _NEG)
        out_ref[0, pl.ds(i * BV, BV)] = logits[0, :]
        bm = jnp.max(logits)
        prev = jnp.where(i == 0, _NEG, m_ref[0])
        m_ref[0] = jnp.maximum(prev, bm)

    @pl.when(i == NB)
    def _finalize():
        m = m_ref[0]
        x = out_ref[...]
        s = jnp.sum(jnp.exp(x - m))
        out_ref[...] = x - m - jnp.log(jnp.full((1, VPAD), s, jnp.float32))


def _tc_logsoftmax(partials, W, b2d):
    return pl.pallas_call(
        _tc_kernel,
        grid=(NB + 1,),
        in_specs=[
            pl.BlockSpec((32, EMB), lambda i: (0, 0)),
            pl.BlockSpec((BV, EMB), lambda i: (jnp.minimum(i, NB - 1), 0)),
            pl.BlockSpec((1, BV), lambda i: (0, jnp.minimum(i, NB - 1))),
        ],
        out_specs=pl.BlockSpec((1, VPAD), lambda i: (0, 0)),
        out_shape=jax.ShapeDtypeStruct((1, VOCAB), jnp.float32),
        scratch_shapes=[
            pltpu.SMEM((1,), jnp.float32),
        ],
    )(partials, W, b2d)


def kernel(context_idxs, emb_table, W, b):
    idx = context_idxs.astype(jnp.int32)
    partials = _sc_gather(idx, emb_table)
    return _tc_logsoftmax(partials, W, b.reshape(1, VOCAB))
